# baseline (speedup 1.0000x reference)
import functools

import jax
import jax.numpy as jnp
from jax import lax
from jax.experimental import pallas as pl
from jax.experimental.pallas import tpu as pltpu
from jax.experimental.pallas import tpu_sc as plsc

_B = 16384
_EMB = 64
_ROW = _EMB + 1
_NC = 2
_NS = 16
_NW = _NC * _NS
_CHUNK = _B // _NW
_GCHUNK = 128
_NGC = _CHUNK // _GCHUNK
_GPC = _GCHUNK // 16

_UV0, _UV1, _UV2, _UV3 = 944, 22, 2, 5
_GV0, _GV1, _GV2 = 1683, 5, 20
_NCU = _UV1 * _UV2 * _UV3
_NCG = _GV1 * _GV2

_CGOFF = _NCU * _ROW
_W1G0OFF = _CGOFF + _NCG * _ROW
_W1COFF = _W1G0OFF + _GV0
_WSMTOT = _W1COFF + _NCG


def _body(idx_h, wu0_h, wg0_h, wsm_h, out_h,
          gidb, uidb, ptb, catb, jobb, sexb, ageb,
          ubuf, gbuf, wsmv, outb, gsems, rsem):
    wid = lax.axis_index("s") * _NC + lax.axis_index("c")
    base = wid * _CHUNK

    def fsl(f):
        return pl.ds(f * _B + base, _CHUNK)

    pltpu.sync_copy(idx_h.at[fsl(0)], gidb)
    pltpu.sync_copy(idx_h.at[fsl(3)], uidb)
    copies = []
    for c in range(_NGC):
        sl = pl.ds(c * _GCHUNK, _GCHUNK)
        copies.append(pltpu.async_copy(
            wg0_h.at[gidb.at[sl]], gbuf.at[sl], gsems.at[c]))
        copies.append(pltpu.async_copy(
            wu0_h.at[uidb.at[sl]], ubuf.at[sl], gsems.at[c]))
    rcopies = [
        pltpu.async_copy(idx_h.at[fsl(1)], ptb, rsem),
        pltpu.async_copy(idx_h.at[fsl(2)], catb, rsem),
        pltpu.async_copy(idx_h.at[fsl(4)], jobb, rsem),
        pltpu.async_copy(idx_h.at[fsl(5)], sexb, rsem),
        pltpu.async_copy(idx_h.at[fsl(6)], ageb, rsem),
        pltpu.async_copy(wsm_h, wsmv, rsem),
    ]
    for cp in rcopies:
        cp.wait()

    rowi = lax.iota(jnp.int32, 16)

    def group(g, carry):
        s0 = g * 16
        sl = pl.ds(s0, 16)
        gv = gidb[sl]
        pv = ptb[sl]
        cv = catb[sl]
        jv = jobb[sl]
        sv = sexb[sl]
        av = ageb[sl]
        cu = jv * (_UV2 * _UV3) + sv * _UV3 + av
        cg = pv * _GV2 + cv
        g1 = (plsc.load_gather(wsmv, [gv + _W1G0OFF])
              + plsc.load_gather(wsmv, [cg + _W1COFF]))
        cub = cu * _ROW
        cgb = cg * _ROW + _CGOFF
        row = rowi + s0
        accs = [jnp.zeros((16,), jnp.float32) for _ in range(4)]
        for d in range(_EMB):
            coloff = (rowi + d) & (_EMB - 1)
            u = (plsc.load_gather(ubuf, [row, coloff])
                 + plsc.load_gather(wsmv, [cub + coloff]))
            gg = (plsc.load_gather(gbuf, [row, coloff])
                  + plsc.load_gather(wsmv, [cgb + coloff]))
            accs[d % 4] = accs[d % 4] + u * gg
        acc = (accs[0] + accs[1]) + (accs[2] + accs[3])
        logit = g1 + acc * (1.0 / 12.0)
        outb[sl] = 5.0 / (1.0 + jnp.exp(-logit))
        return carry

    for c in range(_NGC):
        copies[2 * c].wait()
        copies[2 * c + 1].wait()
        lax.fori_loop(c * _GPC, (c + 1) * _GPC, group, 0)

    pltpu.sync_copy(outb, out_h.at[pl.ds(base, _CHUNK)])


@functools.cache
def _build_fm():
    mesh = plsc.VectorSubcoreMesh(
        core_axis_name="c", subcore_axis_name="s",
        num_cores=_NC, num_subcores=_NS)
    return pl.kernel(
        _body,
        out_type=jax.ShapeDtypeStruct((_B,), jnp.float32),
        mesh=mesh,
        compiler_params=pltpu.CompilerParams(
            needs_layout_passes=False, use_tc_tiling_on_sc=False),
        scratch_types=[
            pltpu.VMEM((_CHUNK,), jnp.int32),
            pltpu.VMEM((_CHUNK,), jnp.int32),
            pltpu.VMEM((_CHUNK,), jnp.int32),
            pltpu.VMEM((_CHUNK,), jnp.int32),
            pltpu.VMEM((_CHUNK,), jnp.int32),
            pltpu.VMEM((_CHUNK,), jnp.int32),
            pltpu.VMEM((_CHUNK,), jnp.int32),
            pltpu.VMEM((_CHUNK, _EMB), jnp.float32),
            pltpu.VMEM((_CHUNK, _EMB), jnp.float32),
            pltpu.VMEM((_WSMTOT,), jnp.float32),
            pltpu.VMEM((_CHUNK,), jnp.float32),
            pltpu.SemaphoreType.DMA((_NGC,)),
            pltpu.SemaphoreType.DMA,
        ],
    )


@jax.jit
def kernel(gid, pubtime, category, uid, job, sex, age,
           W1g0, W1g1, W1g2, Wu0, Wu1, Wu2, Wu3, Wg0, Wg1, Wg2):
    i32 = jnp.int32
    idx = jnp.stack(
        [gid, pubtime, category, uid, job, sex, age]).astype(i32).reshape(-1)
    wu0p = Wu0
    wg0p = Wg0
    cuf = (Wu1[:, None, None, :] + Wu2[None, :, None, :]
           + Wu3[None, None, :, :]).reshape(_NCU, _EMB)
    cgf = (Wg1[:, None, :] + Wg2[None, :, :]).reshape(_NCG, _EMB)
    wsm = jnp.concatenate([
        jnp.pad(cuf, ((0, 0), (0, 1))).reshape(-1),
        jnp.pad(cgf, ((0, 0), (0, 1))).reshape(-1),
        W1g0[:, 0],
        (W1g1[:, 0][:, None] + W1g2[:, 0][None, :]).reshape(-1),
    ])

    out = _build_fm()(idx, wu0p, wg0p, wsm)
    return out[:, None]


# speedup vs baseline: 1.0412x; 1.0412x over previous
import functools

import jax
import jax.numpy as jnp
from jax import lax
from jax.experimental import pallas as pl
from jax.experimental.pallas import tpu as pltpu
from jax.experimental.pallas import tpu_sc as plsc

_B = 16384
_EMB = 64
_ROW = _EMB + 1
_NC = 2
_NS = 16
_NW = _NC * _NS
_CHUNK = _B // _NW
_GCHUNK = 128
_NGC = _CHUNK // _GCHUNK
_GPC = _GCHUNK // 16

_UV0, _UV1, _UV2, _UV3 = 944, 22, 2, 5
_GV0, _GV1, _GV2 = 1683, 5, 20
_NCU = _UV1 * _UV2 * _UV3
_NCG = _GV1 * _GV2

_CGOFF = _NCU * _ROW
_W1G0OFF = _CGOFF + _NCG * _ROW
_W1COFF = _W1G0OFF + _GV0
_WSMTOT = _W1COFF + _NCG


def _body(gid_h, pt_h, cat_h, uid_h, job_h, sex_h, age_h,
          wu0_h, wg0_h, wsm_h, out_h,
          gidb, uidb, ptb, catb, jobb, sexb, ageb,
          ubuf, gbuf, wsmv, outb, gsems, rsem):
    wid = lax.axis_index("s") * _NC + lax.axis_index("c")
    base = wid * _CHUNK
    bsl = pl.ds(base, _CHUNK)

    pltpu.sync_copy(gid_h.at[bsl], gidb)
    pltpu.sync_copy(uid_h.at[bsl], uidb)
    copies = []
    for c in range(_NGC):
        sl = pl.ds(c * _GCHUNK, _GCHUNK)
        copies.append(pltpu.async_copy(
            wg0_h.at[gidb.at[sl]], gbuf.at[sl], gsems.at[c]))
        copies.append(pltpu.async_copy(
            wu0_h.at[uidb.at[sl]], ubuf.at[sl], gsems.at[c]))
    rcopies = [
        pltpu.async_copy(pt_h.at[bsl], ptb, rsem),
        pltpu.async_copy(cat_h.at[bsl], catb, rsem),
        pltpu.async_copy(job_h.at[bsl], jobb, rsem),
        pltpu.async_copy(sex_h.at[bsl], sexb, rsem),
        pltpu.async_copy(age_h.at[bsl], ageb, rsem),
        pltpu.async_copy(wsm_h, wsmv, rsem),
    ]
    for cp in rcopies:
        cp.wait()

    rowi = lax.iota(jnp.int32, 16)

    def group(g, carry):
        s0 = g * 16
        sl = pl.ds(s0, 16)
        gv = gidb[sl]
        pv = ptb[sl]
        cv = catb[sl]
        jv = jobb[sl]
        sv = sexb[sl]
        av = ageb[sl]
        cu = jv * (_UV2 * _UV3) + sv * _UV3 + av
        cg = pv * _GV2 + cv
        g1 = (plsc.load_gather(wsmv, [gv + _W1G0OFF])
              + plsc.load_gather(wsmv, [cg + _W1COFF]))
        cub = cu * _ROW
        cgb = cg * _ROW + _CGOFF
        row = rowi + s0
        accs = [jnp.zeros((16,), jnp.float32) for _ in range(4)]
        for d in range(_EMB):
            coloff = (rowi + d) & (_EMB - 1)
            u = (plsc.load_gather(ubuf, [row, coloff])
                 + plsc.load_gather(wsmv, [cub + coloff]))
            gg = (plsc.load_gather(gbuf, [row, coloff])
                  + plsc.load_gather(wsmv, [cgb + coloff]))
            accs[d % 4] = accs[d % 4] + u * gg
        acc = (accs[0] + accs[1]) + (accs[2] + accs[3])
        logit = g1 + acc * (1.0 / 12.0)
        outb[sl] = 5.0 / (1.0 + jnp.exp(-logit))
        return carry

    for c in range(_NGC):
        copies[2 * c].wait()
        copies[2 * c + 1].wait()
        lax.fori_loop(c * _GPC, (c + 1) * _GPC, group, 0)

    pltpu.sync_copy(outb, out_h.at[pl.ds(base, _CHUNK)])


@functools.cache
def _build_fm():
    mesh = plsc.VectorSubcoreMesh(
        core_axis_name="c", subcore_axis_name="s",
        num_cores=_NC, num_subcores=_NS)
    return pl.kernel(
        _body,
        out_type=jax.ShapeDtypeStruct((_B,), jnp.float32),
        mesh=mesh,
        compiler_params=pltpu.CompilerParams(
            needs_layout_passes=False, use_tc_tiling_on_sc=False),
        scratch_types=[
            pltpu.VMEM((_CHUNK,), jnp.int32),
            pltpu.VMEM((_CHUNK,), jnp.int32),
            pltpu.VMEM((_CHUNK,), jnp.int32),
            pltpu.VMEM((_CHUNK,), jnp.int32),
            pltpu.VMEM((_CHUNK,), jnp.int32),
            pltpu.VMEM((_CHUNK,), jnp.int32),
            pltpu.VMEM((_CHUNK,), jnp.int32),
            pltpu.VMEM((_CHUNK, _EMB), jnp.float32),
            pltpu.VMEM((_CHUNK, _EMB), jnp.float32),
            pltpu.VMEM((_WSMTOT,), jnp.float32),
            pltpu.VMEM((_CHUNK,), jnp.float32),
            pltpu.SemaphoreType.DMA((_NGC,)),
            pltpu.SemaphoreType.DMA,
        ],
    )


@jax.jit
def kernel(gid, pubtime, category, uid, job, sex, age,
           W1g0, W1g1, W1g2, Wu0, Wu1, Wu2, Wu3, Wg0, Wg1, Wg2):
    i32 = jnp.int32
    cuf = (Wu1[:, None, None, :] + Wu2[None, :, None, :]
           + Wu3[None, None, :, :]).reshape(_NCU, _EMB)
    cgf = (Wg1[:, None, :] + Wg2[None, :, :]).reshape(_NCG, _EMB)
    wsm = jnp.concatenate([
        jnp.pad(cuf, ((0, 0), (0, 1))).reshape(-1),
        jnp.pad(cgf, ((0, 0), (0, 1))).reshape(-1),
        W1g0[:, 0],
        (W1g1[:, 0][:, None] + W1g2[:, 0][None, :]).reshape(-1),
    ])

    out = _build_fm()(
        gid.astype(i32), pubtime.astype(i32), category.astype(i32),
        uid.astype(i32), job.astype(i32), sex.astype(i32), age.astype(i32),
        Wu0, Wg0, wsm)
    return out[:, None]


# speedup vs baseline: 1.4339x; 1.3772x over previous
import functools

import jax
import jax.numpy as jnp
from jax import lax
from jax.experimental import pallas as pl
from jax.experimental.pallas import tpu as pltpu
from jax.experimental.pallas import tpu_sc as plsc

_B = 16384
_EMB = 64
_ROW = _EMB + 1
_NC = 2
_NS = 16
_NW = _NC * _NS
_CHUNK = _B // _NW
_GCHUNK = 128
_NGC = _CHUNK // _GCHUNK
_GPC = _GCHUNK // 16

_UV0, _UV1, _UV2, _UV3 = 944, 22, 2, 5
_GV0, _GV1, _GV2 = 1683, 5, 20
_NCU = _UV1 * _UV2 * _UV3
_NCG = _GV1 * _GV2

_CGOFF = _NCU * _ROW
_W1G0OFF = _CGOFF + _NCG * _ROW
_W1COFF = _W1G0OFF + _GV0
_WSMTOT = _W1COFF + _NCG


def _body(gid_h, pt_h, cat_h, uid_h, job_h, sex_h, age_h,
          wu0_h, wg0_h, wsm_h, out_h,
          gidb, uidb, ptb, catb, jobb, sexb, ageb,
          ubuf, gbuf, wsmv, outb, gsems, rsem):
    wid = lax.axis_index("s") * _NC + lax.axis_index("c")
    base = wid * _CHUNK
    bsl = pl.ds(base, _CHUNK)

    pltpu.sync_copy(gid_h.at[bsl], gidb)
    pltpu.sync_copy(uid_h.at[bsl], uidb)
    copies = []
    for c in range(_NGC):
        sl = pl.ds(c * _GCHUNK, _GCHUNK)
        copies.append(pltpu.async_copy(
            wg0_h.at[gidb.at[sl]], gbuf.at[sl], gsems.at[c]))
        copies.append(pltpu.async_copy(
            wu0_h.at[uidb.at[sl]], ubuf.at[sl], gsems.at[c]))
    rcopies = [
        pltpu.async_copy(pt_h.at[bsl], ptb, rsem),
        pltpu.async_copy(cat_h.at[bsl], catb, rsem),
        pltpu.async_copy(job_h.at[bsl], jobb, rsem),
        pltpu.async_copy(sex_h.at[bsl], sexb, rsem),
        pltpu.async_copy(age_h.at[bsl], ageb, rsem),
        pltpu.async_copy(wsm_h, wsmv, rsem),
    ]
    for cp in rcopies:
        cp.wait()
    for cp in copies:
        cp.wait()

    rowi = lax.iota(jnp.int32, 16)

    def group(g, carry):
        s0 = g * 16
        sl = pl.ds(s0, 16)
        gv = gidb[sl]
        pv = ptb[sl]
        cv = catb[sl]
        jv = jobb[sl]
        sv = sexb[sl]
        av = ageb[sl]
        cu = jv * (_UV2 * _UV3) + sv * _UV3 + av
        cg = pv * _GV2 + cv
        g1 = (plsc.load_gather(wsmv, [gv + _W1G0OFF])
              + plsc.load_gather(wsmv, [cg + _W1COFF]))
        cub = cu * _ROW
        cgb = cg * _ROW + _CGOFF
        row = rowi + s0
        accs = [jnp.zeros((16,), jnp.float32) for _ in range(4)]
        for d in range(_EMB):
            coloff = (rowi + d) & (_EMB - 1)
            u = (plsc.load_gather(ubuf, [row, coloff])
                 + plsc.load_gather(wsmv, [cub + coloff]))
            gg = (plsc.load_gather(gbuf, [row, coloff])
                  + plsc.load_gather(wsmv, [cgb + coloff]))
            accs[d % 4] = accs[d % 4] + u * gg
        acc = (accs[0] + accs[1]) + (accs[2] + accs[3])
        logit = g1 + acc * (1.0 / 12.0)
        outb[sl] = 5.0 / (1.0 + jnp.exp(-logit))
        return carry

    lax.fori_loop(0, _NGC * _GPC, group, 0)

    pltpu.sync_copy(outb, out_h.at[pl.ds(base, _CHUNK)])


@functools.cache
def _build_fm():
    mesh = plsc.VectorSubcoreMesh(
        core_axis_name="c", subcore_axis_name="s",
        num_cores=_NC, num_subcores=_NS)
    return pl.kernel(
        _body,
        out_type=jax.ShapeDtypeStruct((_B,), jnp.float32),
        mesh=mesh,
        compiler_params=pltpu.CompilerParams(
            needs_layout_passes=False, use_tc_tiling_on_sc=False),
        scratch_types=[
            pltpu.VMEM((_CHUNK,), jnp.int32),
            pltpu.VMEM((_CHUNK,), jnp.int32),
            pltpu.VMEM((_CHUNK,), jnp.int32),
            pltpu.VMEM((_CHUNK,), jnp.int32),
            pltpu.VMEM((_CHUNK,), jnp.int32),
            pltpu.VMEM((_CHUNK,), jnp.int32),
            pltpu.VMEM((_CHUNK,), jnp.int32),
            pltpu.VMEM((_CHUNK, _EMB), jnp.float32),
            pltpu.VMEM((_CHUNK, _EMB), jnp.float32),
            pltpu.VMEM((_WSMTOT,), jnp.float32),
            pltpu.VMEM((_CHUNK,), jnp.float32),
            pltpu.SemaphoreType.DMA((_NGC,)),
            pltpu.SemaphoreType.DMA,
        ],
    )


@jax.jit
def kernel(gid, pubtime, category, uid, job, sex, age,
           W1g0, W1g1, W1g2, Wu0, Wu1, Wu2, Wu3, Wg0, Wg1, Wg2):
    i32 = jnp.int32
    cuf = (Wu1[:, None, None, :] + Wu2[None, :, None, :]
           + Wu3[None, None, :, :]).reshape(_NCU, _EMB)
    cgf = (Wg1[:, None, :] + Wg2[None, :, :]).reshape(_NCG, _EMB)
    wsm = jnp.concatenate([
        jnp.pad(cuf, ((0, 0), (0, 1))).reshape(-1),
        jnp.pad(cgf, ((0, 0), (0, 1))).reshape(-1),
        W1g0[:, 0],
        (W1g1[:, 0][:, None] + W1g2[:, 0][None, :]).reshape(-1),
    ])

    out = _build_fm()(
        gid.astype(i32), pubtime.astype(i32), category.astype(i32),
        uid.astype(i32), job.astype(i32), sex.astype(i32), age.astype(i32),
        Wu0, Wg0, wsm)
    return out[:, None]
